# Initial kernel scaffold; baseline (speedup 1.0000x reference)
import functools

import jax
import jax.numpy as jnp
from jax import lax
from jax.experimental import pallas as pl
from jax.experimental.pallas import tpu as pltpu
from jax.experimental.pallas import tpu_sc as plsc

EMBEDDING_DIM = 64
NUM_WORKERS = 32
CHUNK = 512


def _gather_kernel_body(n_chunks, b_per_w,
                        table_hbm, ids_hbm, out_hbm, idx_v, rows_v, sem):
    wid = lax.axis_index("s") * 2 + lax.axis_index("c")
    base = wid * b_per_w

    def body(g, carry):
        off = base + g * CHUNK
        pltpu.sync_copy(ids_hbm.at[pl.ds(off, CHUNK)], idx_v)
        pltpu.async_copy(table_hbm.at[idx_v], rows_v, sem).wait()
        pltpu.sync_copy(rows_v, out_hbm.at[pl.ds(off, CHUNK)])
        return carry

    lax.fori_loop(0, n_chunks, body, 0)


def kernel(ids, table):
    batch, seq = ids.shape
    dim = table.shape[1]
    b_total = batch * seq
    b_per_w = b_total // NUM_WORKERS
    n_chunks = b_per_w // CHUNK
    ids_flat = ids.reshape(-1)

    mesh = plsc.VectorSubcoreMesh(core_axis_name="c", subcore_axis_name="s")
    gather = pl.kernel(
        functools.partial(_gather_kernel_body, n_chunks, b_per_w),
        out_type=jax.ShapeDtypeStruct((b_total, dim), jnp.float32),
        mesh=mesh,
        scratch_types=[
            pltpu.VMEM((CHUNK,), jnp.int32),
            pltpu.VMEM((CHUNK, dim), jnp.float32),
            pltpu.SemaphoreType.DMA,
        ],
    )
    out = gather(table, ids_flat)
    return out.reshape(batch, seq, dim)


# speedup vs baseline: 1.7990x; 1.7990x over previous
import functools

import jax
import jax.numpy as jnp
from jax import lax
from jax.experimental import pallas as pl
from jax.experimental.pallas import tpu as pltpu
from jax.experimental.pallas import tpu_sc as plsc

EMBEDDING_DIM = 64
NUM_WORKERS = 32
CHUNK = 512


def _gather_kernel_body(n_chunks, b_per_w,
                        table_hbm, ids_hbm, out_hbm, idx_v, rows_v, sem):
    wid = lax.axis_index("s") * 2 + lax.axis_index("c")
    base = wid * b_per_w

    def body(g, carry):
        off = base + g * CHUNK
        pltpu.sync_copy(ids_hbm.at[pl.ds(off, CHUNK)], idx_v)
        pltpu.async_copy(table_hbm.at[idx_v], rows_v, sem).wait()
        pltpu.sync_copy(rows_v, out_hbm.at[pl.ds(off, CHUNK)])
        return carry

    lax.fori_loop(0, n_chunks, body, 0)


def kernel(ids, table):
    batch, seq = ids.shape
    dim = table.shape[1]
    b_total = batch * seq
    b_per_w = b_total // NUM_WORKERS
    n_chunks = b_per_w // CHUNK
    ids_flat = ids.reshape(-1)

    mesh = plsc.VectorSubcoreMesh(core_axis_name="c", subcore_axis_name="s")
    gather = pl.kernel(
        functools.partial(_gather_kernel_body, n_chunks, b_per_w),
        out_type=jax.ShapeDtypeStruct((b_total, dim), jnp.float32),
        mesh=mesh,
        scratch_types=[
            pltpu.VMEM((CHUNK,), jnp.int32),
            pltpu.VMEM((CHUNK, dim), jnp.float32),
            pltpu.SemaphoreType.DMA,
        ],
        compiler_params=pltpu.CompilerParams(use_tc_tiling_on_sc=False),
    )
    out = gather(table, ids_flat)
    return out.reshape(batch, seq, dim)


# speedup vs baseline: 1.8674x; 1.0380x over previous
import functools

import jax
import jax.numpy as jnp
from jax import lax
from jax.experimental import pallas as pl
from jax.experimental.pallas import tpu as pltpu
from jax.experimental.pallas import tpu_sc as plsc

NUM_WORKERS = 32
CHUNK = 800


def _gather_body(n_chunks, b_per_w,
                 table_hbm, ids_hbm, out_hbm,
                 ids_all, rows0, rows1,
                 gsem0, gsem1, osem0, osem1):
    wid = lax.axis_index("s") * 2 + lax.axis_index("c")
    base = wid * b_per_w
    pltpu.sync_copy(ids_hbm.at[pl.ds(base, b_per_w)], ids_all)

    rows = (rows0, rows1)
    gsem = (gsem0, gsem1)
    osem = (osem0, osem1)

    def start_gather(chunk_i, buf, sem):
        off = chunk_i * CHUNK
        pltpu.async_copy(table_hbm.at[ids_all.at[pl.ds(off, CHUNK)]], buf, sem)

    def start_write(chunk_i, buf, sem):
        off = base + chunk_i * CHUNK
        pltpu.async_copy(buf, out_hbm.at[pl.ds(off, CHUNK)], sem)

    def wait(sem, src, dst):
        pltpu.make_async_copy(src, dst, sem).wait()

    last = n_chunks - 1

    start_gather(0, rows0, gsem0)
    wait(gsem0, table_hbm.at[ids_all.at[pl.ds(0, CHUNK)]], rows0)
    start_gather(1, rows1, gsem1)
    start_write(0, rows0, osem0)

    wait(gsem1, table_hbm.at[ids_all.at[pl.ds(0, CHUNK)]], rows1)
    wait(osem0, rows0, out_hbm.at[pl.ds(base, CHUNK)])
    start_gather(2, rows0, gsem0)
    start_write(1, rows1, osem1)

    def body(gg, carry):
        g = 2 + 2 * gg
        for b in range(2):
            gc = g + b
            p = b
            wait(gsem[p], table_hbm.at[ids_all.at[pl.ds(0, CHUNK)]], rows[p])
            wait(osem[1 - p], rows[1 - p],
                 out_hbm.at[pl.ds(base, CHUNK)])
            nxt = jnp.minimum(gc + 1, last)
            start_gather(nxt, rows[1 - p], gsem[1 - p])
            start_write(gc, rows[p], osem[p])
        return carry

    lax.fori_loop(0, (n_chunks - 2) // 2, body, 0)

    wait(gsem[0], table_hbm.at[ids_all.at[pl.ds(0, CHUNK)]], rows[0])
    wait(osem[1], rows[1], out_hbm.at[pl.ds(base, CHUNK)])


def kernel(ids, table):
    batch, seq = ids.shape
    dim = table.shape[1]
    b_total = batch * seq
    b_per_w = b_total // NUM_WORKERS
    n_chunks = b_per_w // CHUNK
    ids_flat = ids.reshape(-1)

    mesh = plsc.VectorSubcoreMesh(core_axis_name="c", subcore_axis_name="s")
    gather = pl.kernel(
        functools.partial(_gather_body, n_chunks, b_per_w),
        out_type=jax.ShapeDtypeStruct((b_total, dim), jnp.float32),
        mesh=mesh,
        scratch_types=[
            pltpu.VMEM((b_per_w,), jnp.int32),
            pltpu.VMEM((CHUNK, dim), jnp.float32),
            pltpu.VMEM((CHUNK, dim), jnp.float32),
            pltpu.SemaphoreType.DMA,
            pltpu.SemaphoreType.DMA,
            pltpu.SemaphoreType.DMA,
            pltpu.SemaphoreType.DMA,
        ],
        compiler_params=pltpu.CompilerParams(use_tc_tiling_on_sc=False),
    )
    out = gather(table, ids_flat)
    return out.reshape(batch, seq, dim)
